# baseline (speedup 1.0000x reference)
import functools

import jax
import jax.numpy as jnp
from jax import lax
from jax.experimental import pallas as pl
from jax.experimental.pallas import tpu as pltpu
from jax.experimental.pallas import tpu_sc as plsc

N_DST = 10000
D = 128
NC = 2
NS = 16
NW = NC * NS
CHUNK = 80
ACC_ROWS = 10240
ROWS_PER_TILE = ACC_ROWS // NS
E_PAD = 327680


def _sc_partial_sums(node, src, dst, zeros):
    E = src.shape[0]
    per_tile = E // NW
    m = per_tile // CHUNK
    n_pairs = m // 2 - 1

    mesh = plsc.VectorSubcoreMesh(core_axis_name="c", subcore_axis_name="s")

    @functools.partial(
        pl.kernel,
        mesh=mesh,
        out_type=jax.ShapeDtypeStruct((NC * ACC_ROWS, D), jnp.float32),
        scratch_types=[
            pltpu.VMEM((CHUNK,), jnp.int32),
            pltpu.VMEM((CHUNK,), jnp.int32),
            pltpu.VMEM((CHUNK,), jnp.int32),
            pltpu.VMEM((CHUNK,), jnp.int32),
            pltpu.VMEM((CHUNK, D), jnp.float32),
            pltpu.VMEM((CHUNK, D), jnp.float32),
            pltpu.VMEM_SHARED((ACC_ROWS, D), jnp.float32),
            pltpu.SemaphoreType.DMA,
            pltpu.SemaphoreType.DMA,
            pltpu.SemaphoreType.DMA,
            pltpu.SemaphoreType.DMA,
            pltpu.SemaphoreType.DMA,
        ],
    )
    def k(node_hbm, src_hbm, dst_hbm, zeros_hbm, out_hbm,
          src_a, dst_a, src_b, dst_b, rows_a, rows_b,
          acc, sem_a, sem_b, sem_s, sem_ia, sem_ib):
        c = lax.axis_index("c")
        s = lax.axis_index("s")
        wid = s * NC + c

        r0 = s * ROWS_PER_TILE
        pltpu.sync_copy(zeros_hbm.at[pl.ds(r0, ROWS_PER_TILE)],
                        acc.at[pl.ds(r0, ROWS_PER_TILE)])
        plsc.subcore_barrier()

        base0 = wid * per_tile

        def idx_start(chunk, src_c, dst_c, sem):
            e = base0 + chunk * CHUNK
            pltpu.async_copy(src_hbm.at[pl.ds(e, CHUNK)], src_c, sem)
            pltpu.async_copy(dst_hbm.at[pl.ds(e, CHUNK)], dst_c, sem)

        def idx_wait(chunk, src_c, dst_c, sem):
            e = base0 + chunk * CHUNK
            pltpu.make_async_copy(src_hbm.at[pl.ds(e, CHUNK)], src_c,
                                  sem).wait()
            pltpu.make_async_copy(dst_hbm.at[pl.ds(e, CHUNK)], dst_c,
                                  sem).wait()

        def gather_start(src_c, rows_v, sem):
            pltpu.async_copy(node_hbm.at[src_c], rows_v, sem)

        def gather_wait(src_c, rows_v, sem):
            pltpu.make_async_copy(node_hbm.at[src_c], rows_v, sem).wait()

        def scatter_start(dst_c, rows_v):
            pltpu.async_copy(rows_v, acc.at[dst_c], sem_s, add=True)

        def scatter_wait(dst_c, rows_v):
            pltpu.make_async_copy(rows_v, acc.at[dst_c], sem_s).wait()

        def scatter_sync(dst_c, rows_v):
            pltpu.sync_copy(rows_v, acc.at[dst_c], add=True)

        idx_start(0, src_a, dst_a, sem_ia)
        idx_wait(0, src_a, dst_a, sem_ia)
        gather_start(src_a, rows_a, sem_a)

        def body(i, carry):
            idx_start(2 * i + 1, src_b, dst_b, sem_ib)
            gather_wait(src_a, rows_a, sem_a)
            scatter_sync(dst_a, rows_a)
            idx_wait(2 * i + 1, src_b, dst_b, sem_ib)
            gather_start(src_b, rows_b, sem_b)
            idx_start(2 * i + 2, src_a, dst_a, sem_ia)
            idx_wait(2 * i + 2, src_a, dst_a, sem_ia)
            gather_start(src_a, rows_a, sem_a)
            gather_wait(src_b, rows_b, sem_b)
            scatter_sync(dst_b, rows_b)
            return carry

        lax.fori_loop(0, n_pairs, body, 0)
        idx_start(m - 1, src_b, dst_b, sem_ib)
        idx_wait(m - 1, src_b, dst_b, sem_ib)
        gather_start(src_b, rows_b, sem_b)
        gather_wait(src_a, rows_a, sem_a)
        scatter_sync(dst_a, rows_a)
        gather_wait(src_b, rows_b, sem_b)
        scatter_sync(dst_b, rows_b)
        plsc.subcore_barrier()

        pltpu.sync_copy(acc.at[pl.ds(r0, ROWS_PER_TILE)],
                        out_hbm.at[pl.ds(c * ACC_ROWS + r0, ROWS_PER_TILE)])

    return k(node, src, dst, zeros)


def _combine(partials):
    R = 400

    def body(a_ref, b_ref, o_ref):
        o_ref[...] = a_ref[...] + b_ref[...]

    return pl.pallas_call(
        body,
        grid=(N_DST // R,),
        in_specs=[pl.BlockSpec((R, D), lambda i: (i, 0)),
                  pl.BlockSpec((R, D), lambda i: (i, 0))],
        out_specs=pl.BlockSpec((R, D), lambda i: (i, 0)),
        out_shape=jax.ShapeDtypeStruct((N_DST, D), jnp.float32),
    )(partials[:N_DST], partials[ACC_ROWS:ACC_ROWS + N_DST])


def kernel(node, edge_index):
    ei = edge_index.astype(jnp.int32)
    E = ei.shape[1]
    pad_w = (E_PAD - E) // NW
    pad_src = jnp.zeros((NW, pad_w), jnp.int32)
    pad_dst = jnp.broadcast_to(
        N_DST + jnp.arange(pad_w, dtype=jnp.int32), (NW, pad_w))
    src = jnp.concatenate([ei[0].reshape(NW, -1), pad_src], axis=1).reshape(-1)
    dst = jnp.concatenate([ei[1].reshape(NW, -1), pad_dst], axis=1).reshape(-1)
    zeros = jnp.zeros((ACC_ROWS, D), jnp.float32)
    partials = _sc_partial_sums(node, src, dst, zeros)
    return _combine(partials)


# speedup vs baseline: 2.5647x; 2.5647x over previous
import functools

import jax
import jax.numpy as jnp
from jax import lax
from jax.experimental import pallas as pl
from jax.experimental.pallas import tpu as pltpu
from jax.experimental.pallas import tpu_sc as plsc

N_DST = 10000
D = 128
NC = 2
NS = 16
NW = NC * NS
CHUNK = 128
TAIL = 16
ACC_ROWS = 10240
ROWS_PER_TILE = ACC_ROWS // NS


def _sc_partial_sums(node, src, dst, zeros):
    E = src.shape[0]
    per_tile = E // NW
    m = per_tile // CHUNK
    n_pairs = m // 2 - 1

    mesh = plsc.VectorSubcoreMesh(core_axis_name="c", subcore_axis_name="s")

    @functools.partial(
        pl.kernel,
        mesh=mesh,
        out_type=jax.ShapeDtypeStruct((NC * ACC_ROWS, D), jnp.float32),
        scratch_types=[
            pltpu.VMEM((CHUNK,), jnp.int32),
            pltpu.VMEM((CHUNK,), jnp.int32),
            pltpu.VMEM((CHUNK,), jnp.int32),
            pltpu.VMEM((CHUNK,), jnp.int32),
            pltpu.VMEM((TAIL,), jnp.int32),
            pltpu.VMEM((TAIL,), jnp.int32),
            pltpu.VMEM((CHUNK, D), jnp.float32),
            pltpu.VMEM((CHUNK, D), jnp.float32),
            pltpu.VMEM((TAIL, D), jnp.float32),
            pltpu.VMEM_SHARED((ACC_ROWS, D), jnp.float32),
            pltpu.SemaphoreType.DMA,
            pltpu.SemaphoreType.DMA,
            pltpu.SemaphoreType.DMA,
            pltpu.SemaphoreType.DMA,
            pltpu.SemaphoreType.DMA,
        ],
    )
    def k(node_hbm, src_hbm, dst_hbm, zeros_hbm, out_hbm,
          src_a, dst_a, src_b, dst_b, src_t, dst_t, rows_a, rows_b, rows_t,
          acc, sem_a, sem_b, sem_s, sem_ia, sem_ib):
        c = lax.axis_index("c")
        s = lax.axis_index("s")
        wid = s * NC + c

        r0 = s * ROWS_PER_TILE
        pltpu.sync_copy(zeros_hbm.at[pl.ds(r0, ROWS_PER_TILE)],
                        acc.at[pl.ds(r0, ROWS_PER_TILE)])
        plsc.subcore_barrier()

        base0 = wid * per_tile

        def idx_start(chunk, src_c, dst_c, sem):
            e = base0 + chunk * CHUNK
            pltpu.async_copy(src_hbm.at[pl.ds(e, CHUNK)], src_c, sem)
            pltpu.async_copy(dst_hbm.at[pl.ds(e, CHUNK)], dst_c, sem)

        def idx_wait(chunk, src_c, dst_c, sem):
            e = base0 + chunk * CHUNK
            pltpu.make_async_copy(src_hbm.at[pl.ds(e, CHUNK)], src_c,
                                  sem).wait()
            pltpu.make_async_copy(dst_hbm.at[pl.ds(e, CHUNK)], dst_c,
                                  sem).wait()

        def gather_start(src_c, rows_v, sem):
            pltpu.async_copy(node_hbm.at[src_c], rows_v, sem)

        def gather_wait(src_c, rows_v, sem):
            pltpu.make_async_copy(node_hbm.at[src_c], rows_v, sem).wait()

        def scatter_sync(dst_c, rows_v):
            pltpu.sync_copy(rows_v, acc.at[dst_c], add=True)

        idx_start(0, src_a, dst_a, sem_ia)
        idx_wait(0, src_a, dst_a, sem_ia)
        gather_start(src_a, rows_a, sem_a)

        def body(i, carry):
            idx_start(2 * i + 1, src_b, dst_b, sem_ib)
            gather_wait(src_a, rows_a, sem_a)
            scatter_sync(dst_a, rows_a)
            idx_wait(2 * i + 1, src_b, dst_b, sem_ib)
            gather_start(src_b, rows_b, sem_b)
            idx_start(2 * i + 2, src_a, dst_a, sem_ia)
            idx_wait(2 * i + 2, src_a, dst_a, sem_ia)
            gather_start(src_a, rows_a, sem_a)
            gather_wait(src_b, rows_b, sem_b)
            scatter_sync(dst_b, rows_b)
            return carry

        lax.fori_loop(0, n_pairs, body, 0)
        idx_start(m - 1, src_b, dst_b, sem_ib)
        idx_wait(m - 1, src_b, dst_b, sem_ib)
        gather_start(src_b, rows_b, sem_b)
        et = base0 + m * CHUNK
        pltpu.sync_copy(src_hbm.at[pl.ds(et, TAIL)], src_t)
        pltpu.sync_copy(dst_hbm.at[pl.ds(et, TAIL)], dst_t)
        gather_wait(src_a, rows_a, sem_a)
        pltpu.async_copy(node_hbm.at[src_t], rows_t, sem_a)
        scatter_sync(dst_a, rows_a)
        gather_wait(src_b, rows_b, sem_b)
        scatter_sync(dst_b, rows_b)
        pltpu.make_async_copy(node_hbm.at[src_t], rows_t, sem_a).wait()
        scatter_sync(dst_t, rows_t)
        plsc.subcore_barrier()

        pltpu.sync_copy(acc.at[pl.ds(r0, ROWS_PER_TILE)],
                        out_hbm.at[pl.ds(c * ACC_ROWS + r0, ROWS_PER_TILE)])

    return k(node, src, dst, zeros)


def _combine(partials):
    R = 400

    def body(a_ref, b_ref, o_ref):
        o_ref[...] = a_ref[...] + b_ref[...]

    return pl.pallas_call(
        body,
        grid=(N_DST // R,),
        in_specs=[pl.BlockSpec((R, D), lambda i: (i, 0)),
                  pl.BlockSpec((R, D), lambda i: (i, 0))],
        out_specs=pl.BlockSpec((R, D), lambda i: (i, 0)),
        out_shape=jax.ShapeDtypeStruct((N_DST, D), jnp.float32),
    )(partials[:N_DST], partials[ACC_ROWS:ACC_ROWS + N_DST])


def kernel(node, edge_index):
    ei = edge_index.astype(jnp.int32)
    zeros = jnp.zeros((ACC_ROWS, D), jnp.float32)
    partials = _sc_partial_sums(node, ei[0], ei[1], zeros)
    return _combine(partials)
